# baseline (speedup 1.0000x reference)
import functools

import numpy as np
import jax
import jax.numpy as jnp
from jax import lax
from jax.experimental import pallas as pl
from jax.experimental.pallas import tpu as pltpu
from jax.experimental.pallas import tpu_sc as plsc

_B = 128
_V = 100000
_CH = 2944
_NFULL = 33
_TAIL = _V - _NFULL * _CH
_NVT = _TAIL // 16
_NV = _CH // 16
_NC = 2
_NS = 16
_NW = _NC * _NS
_RPW = _B // _NW
_OUTP = 16
_EPS = 1e-10
_IBIG = np.int32(2**31 - 1)

_consts = []


def _threefry2x32_np(k1, k2, x0, x1):
    r1 = (13, 15, 26, 6)
    r2 = (17, 29, 16, 24)

    def rl(x, d):
        return (x << np.uint32(d)) | (x >> np.uint32(32 - d))

    def rounds(x0, x1, rots):
        for r in rots:
            x0 = x0 + x1
            x1 = rl(x1, r)
            x1 = x1 ^ x0
        return x0, x1

    ks0 = np.uint32(k1)
    ks1 = np.uint32(k2)
    ks2 = np.uint32(0x1BD11BDA) ^ ks0 ^ ks1
    x0 = x0 + ks0
    x1 = x1 + ks1
    x0, x1 = rounds(x0, x1, r1)
    x0 = x0 + ks1
    x1 = x1 + (ks2 + np.uint32(1))
    x0, x1 = rounds(x0, x1, r2)
    x0 = x0 + ks2
    x1 = x1 + (ks0 + np.uint32(2))
    x0, x1 = rounds(x0, x1, r1)
    x0 = x0 + ks0
    x1 = x1 + (ks1 + np.uint32(3))
    x0, x1 = rounds(x0, x1, r2)
    x0 = x0 + ks1
    x1 = x1 + (ks2 + np.uint32(4))
    x0, x1 = rounds(x0, x1, r1)
    x0 = x0 + ks2
    x1 = x1 + (ks0 + np.uint32(5))
    return x0, x1


def _noise_recip():
    if not _consts:
        o0, o1 = _threefry2x32_np(np.uint32(0), np.uint32(0),
                                  np.uint32([0]), np.uint32([12345]))
        k1, k2 = o0[0], o1[0]
        idx = np.arange(_B * _V, dtype=np.uint32)
        b0, b1 = _threefry2x32_np(k1, k2, np.zeros_like(idx), idx)
        bits = b0 ^ b1
        fb = (bits >> np.uint32(9)) | np.uint32(0x3F800000)
        u = fb.view(np.float32) - np.float32(1.0)
        n = -np.log1p(-u)
        g = (-np.log(n.astype(np.float64) + _EPS)).astype(np.float32)
        _consts.append(g)
    return _consts[0]


def _sampler_sc_body(row_off, logits_hbm, gumb_hbm, temps_hbm, out_hbm,
                     lbuf0, gbuf0, lbuf1, gbuf1, ltail, gtail, tbuf, obuf,
                     sl0, sg0, sl1, sg1, slt, sgt):
    wid = lax.axis_index("s")
    grp = pl.multiple_of(row_off + 8 * (wid // 2), 8)
    half = wid % 2
    pltpu.sync_copy(temps_hbm, tbuf)
    lane = lax.iota(jnp.int32, 16)
    neg_inf = jnp.float32(-jnp.inf)

    lbufs = [(lbuf0, sl0), (lbuf1, sl1)]
    gbufs = [(gbuf0, sg0), (gbuf1, sg1)]

    def issue(c, par):
        col = pl.multiple_of(c * _CH, 128)
        lb, sl = lbufs[par]
        gb, sg = gbufs[par]
        pltpu.async_copy(logits_hbm.at[pl.ds(grp, 8), pl.ds(col, _CH)],
                         lb, sl)
        for j in range(_RPW):
            off = pl.multiple_of((grp + 4 * half + j) * _V + c * _CH, 8)
            pltpu.async_copy(gumb_hbm.at[pl.ds(off, _CH)],
                             gb.at[pl.ds(j * _CH, _CH)], sg)

    def wait_chunk(c, par):
        col = pl.multiple_of(c * _CH, 128)
        lb, sl = lbufs[par]
        gb, sg = gbufs[par]
        pltpu.make_async_copy(
            logits_hbm.at[pl.ds(grp, 8), pl.ds(col, _CH)], lb, sl).wait()
        for j in range(_RPW):
            pltpu.make_async_copy(
                gumb_hbm.at[pl.ds(0, _CH)], gb.at[pl.ds(j * _CH, _CH)],
                sg).wait()

    def sweep(lb, gb, jrow, j, tv, m_vec, base, nv, carry, gstride):
        def body(i, carry1):
            mv, sb, si = carry1
            sl_ = pl.ds(i * 16, 16)
            l = lb[jrow, sl_]
            a = l + tv * gb[pl.ds(j * gstride + i * 16, 16)]
            s = jnp.exp(a - m_vec)
            vidx = (base + i * 16) + lane
            upd = s > sb
            return (jnp.maximum(mv, a),
                    jnp.where(upd, s, sb),
                    jnp.where(upd, vidx, si))
        return lax.fori_loop(0, nv, body, carry, unroll=5)

    def proc(c, par, st, tail=False):
        if tail:
            lb, gb = ltail, gtail
            base = _NFULL * _CH
            nv = _NVT
        else:
            lb, _ = lbufs[par]
            gb, _ = gbufs[par]
            base = c * _CH
            nv = _NV
        new_st = []
        for j in range(_RPW):
            m_norm, sbest, sidx = st[j]
            row = grp + 4 * half + j
            tv = plsc.load_gather(tbuf, [jnp.full((16,), row, jnp.int32)])
            jrow = 4 * half + j
            mv0 = jnp.full((16,), neg_inf, jnp.float32)
            mv, sb1, si1 = sweep(
                lb, gb, jrow, j, tv, jnp.full((16,), m_norm, jnp.float32),
                base, nv, (mv0, sbest, sidx), nv * 16)
            m_new = jnp.max(mv)

            def no_redo(_, sb1=sb1, si1=si1, m_norm=m_norm):
                return m_norm, sb1, si1

            def redo(_, lb=lb, gb=gb, jrow=jrow, j=j, tv=tv, base=base,
                     nv=nv, mv=mv, m_new=m_new, m_norm=m_norm, sbest=sbest,
                     sidx=sidx):
                sb0 = sbest * jnp.exp(
                    jnp.full((16,), m_norm - m_new, jnp.float32))
                _, sb2, si2 = sweep(
                    lb, gb, jrow, j, tv,
                    jnp.full((16,), m_new, jnp.float32),
                    base, nv, (mv, sb0, sidx), nv * 16)
                return m_new, sb2, si2

            m2, sb2, si2 = lax.cond(m_new > m_norm + jnp.float32(80.0),
                                    redo, no_redo, 0)
            new_st.append((m2, sb2, si2))
        return tuple(new_st)

    st = tuple((neg_inf,
                jnp.zeros((16,), jnp.float32), jnp.zeros((16,), jnp.int32))
               for _ in range(_RPW))

    issue(0, 0)
    issue(1, 1)

    def loop_body(k, st):
        c0 = 2 * k
        wait_chunk(c0, 0)
        st = proc(c0, 0, st)
        issue(c0 + 2, 0)
        wait_chunk(c0 + 1, 1)
        st = proc(c0 + 1, 1, st)
        issue(c0 + 3, 1)
        return st

    st = lax.fori_loop(0, 15, loop_body, st)

    wait_chunk(30, 0)
    st = proc(30, 0, st)
    issue(32, 0)
    wait_chunk(31, 1)
    st = proc(31, 1, st)
    tcol = _NFULL * _CH
    pltpu.async_copy(logits_hbm.at[pl.ds(grp, 8), pl.ds(tcol, _TAIL)],
                     ltail, slt)
    for j in range(_RPW):
        toff = pl.multiple_of((grp + 4 * half + j) * _V + tcol, 8)
        pltpu.async_copy(gumb_hbm.at[pl.ds(toff, _TAIL)],
                         gtail.at[pl.ds(j * _TAIL, _TAIL)], sgt)
    wait_chunk(32, 0)
    st = proc(32, 0, st)
    pltpu.make_async_copy(logits_hbm.at[pl.ds(grp, 8), pl.ds(tcol, _TAIL)],
                          ltail, slt).wait()
    for j in range(_RPW):
        pltpu.make_async_copy(gumb_hbm.at[pl.ds(0, _TAIL)],
                              gtail.at[pl.ds(j * _TAIL, _TAIL)], sgt).wait()
    st = proc(0, 0, st, tail=True)

    tokens = jnp.zeros((16,), jnp.int32)
    ibig = jnp.full((16,), _IBIG, jnp.int32)
    for j in range(_RPW):
        _, sbest, sidx = st[j]
        smax = jnp.full((16,), jnp.max(sbest), jnp.float32)
        stok = jnp.min(jnp.where(sbest == smax, sidx, ibig))
        tokens = jnp.where(lane == j, jnp.full((16,), stok, jnp.int32),
                           tokens)

    obuf[...] = tokens
    pltpu.sync_copy(obuf, out_hbm.at[pl.ds(wid * _OUTP, _OUTP)])


_sampler_cache = []


def _sampler_sc(idx):
    if not _sampler_cache:
        for off in (0, 64):
            _sampler_cache.append(pl.kernel(
                functools.partial(_sampler_sc_body, off),
                out_type=jax.ShapeDtypeStruct((_NS * _OUTP,), jnp.int32),
                mesh=plsc.VectorSubcoreMesh(core_axis_name="c",
                                            subcore_axis_name="s",
                                            num_cores=1, num_subcores=_NS),
            scratch_types=[
                pltpu.VMEM((8, _CH), jnp.float32),
                pltpu.VMEM((_RPW * _CH,), jnp.float32),
                pltpu.VMEM((8, _CH), jnp.float32),
                pltpu.VMEM((_RPW * _CH,), jnp.float32),
                pltpu.VMEM((8, _TAIL), jnp.float32),
                pltpu.VMEM((_RPW * _TAIL,), jnp.float32),
                pltpu.VMEM((_B,), jnp.float32),
                pltpu.VMEM((_OUTP,), jnp.int32),
                pltpu.SemaphoreType.DMA,
                pltpu.SemaphoreType.DMA,
                pltpu.SemaphoreType.DMA,
                pltpu.SemaphoreType.DMA,
                pltpu.SemaphoreType.DMA,
                pltpu.SemaphoreType.DMA,
            ],
            compiler_params=pltpu.CompilerParams(
                needs_layout_passes=False),
            ))
    return _sampler_cache[idx]


def kernel(logits, temperatures):
    gumb = jnp.asarray(_noise_recip())
    lo = _sampler_sc(0)(logits, gumb, temperatures)
    hi = _sampler_sc(1)(logits, gumb, temperatures)
    both = jnp.concatenate([lo, hi]).reshape(2 * _NS, _OUTP)
    return both[:, :_RPW].reshape(_B)


# speedup vs baseline: 1.6349x; 1.6349x over previous
import functools

import numpy as np
import jax
import jax.numpy as jnp
from jax import lax
from jax.experimental import pallas as pl
from jax.experimental.pallas import tpu as pltpu
from jax.experimental.pallas import tpu_sc as plsc

_B = 128
_V = 100000
_CH = 2944
_NFULL = 33
_TAIL = _V - _NFULL * _CH
_NVT = _TAIL // 16
_NV = _CH // 16
_NC = 2
_NS = 16
_NW = _NC * _NS
_RPW = _B // _NW
_OUTP = 16
_EPS = 1e-10
_IBIG = np.int32(2**31 - 1)

_consts = []


def _threefry2x32_np(k1, k2, x0, x1):
    r1 = (13, 15, 26, 6)
    r2 = (17, 29, 16, 24)

    def rl(x, d):
        return (x << np.uint32(d)) | (x >> np.uint32(32 - d))

    def rounds(x0, x1, rots):
        for r in rots:
            x0 = x0 + x1
            x1 = rl(x1, r)
            x1 = x1 ^ x0
        return x0, x1

    ks0 = np.uint32(k1)
    ks1 = np.uint32(k2)
    ks2 = np.uint32(0x1BD11BDA) ^ ks0 ^ ks1
    x0 = x0 + ks0
    x1 = x1 + ks1
    x0, x1 = rounds(x0, x1, r1)
    x0 = x0 + ks1
    x1 = x1 + (ks2 + np.uint32(1))
    x0, x1 = rounds(x0, x1, r2)
    x0 = x0 + ks2
    x1 = x1 + (ks0 + np.uint32(2))
    x0, x1 = rounds(x0, x1, r1)
    x0 = x0 + ks0
    x1 = x1 + (ks1 + np.uint32(3))
    x0, x1 = rounds(x0, x1, r2)
    x0 = x0 + ks1
    x1 = x1 + (ks2 + np.uint32(4))
    x0, x1 = rounds(x0, x1, r1)
    x0 = x0 + ks2
    x1 = x1 + (ks0 + np.uint32(5))
    return x0, x1


def _noise_recip():
    if not _consts:
        o0, o1 = _threefry2x32_np(np.uint32(0), np.uint32(0),
                                  np.uint32([0]), np.uint32([12345]))
        k1, k2 = o0[0], o1[0]
        idx = np.arange(_B * _V, dtype=np.uint32)
        b0, b1 = _threefry2x32_np(k1, k2, np.zeros_like(idx), idx)
        bits = b0 ^ b1
        fb = (bits >> np.uint32(9)) | np.uint32(0x3F800000)
        u = fb.view(np.float32) - np.float32(1.0)
        n = -np.log1p(-u)
        g = (-np.log(n.astype(np.float64) + _EPS)).astype(np.float32)
        _consts.append(g)
    return _consts[0]


def _sampler_sc_body(logits_hbm, gumb_hbm, temps_hbm, out_hbm,
                     lbuf0, gbuf0, lbuf1, gbuf1, ltail, gtail, tbuf, obuf,
                     sl0, sg0, sl1, sg1, slt, sgt):
    wid = lax.axis_index("s") * _NC + lax.axis_index("c")
    grp = pl.multiple_of(8 * (wid // 2), 8)
    half = wid % 2
    pltpu.sync_copy(temps_hbm, tbuf)
    lane = lax.iota(jnp.int32, 16)
    neg_inf = jnp.float32(-jnp.inf)

    lbufs = [(lbuf0, sl0), (lbuf1, sl1)]
    gbufs = [(gbuf0, sg0), (gbuf1, sg1)]

    def issue(c, par):
        col = pl.multiple_of(c * _CH, 128)
        lb, sl = lbufs[par]
        gb, sg = gbufs[par]
        pltpu.async_copy(logits_hbm.at[pl.ds(grp, 8), pl.ds(col, _CH)],
                         lb, sl)
        for j in range(_RPW):
            off = pl.multiple_of((wid * _RPW + j) * _V + c * _CH, 8)
            pltpu.async_copy(gumb_hbm.at[pl.ds(off, _CH)],
                             gb.at[pl.ds(j * _CH, _CH)], sg)

    def wait_chunk(c, par):
        col = pl.multiple_of(c * _CH, 128)
        lb, sl = lbufs[par]
        gb, sg = gbufs[par]
        pltpu.make_async_copy(
            logits_hbm.at[pl.ds(grp, 8), pl.ds(col, _CH)], lb, sl).wait()
        for j in range(_RPW):
            pltpu.make_async_copy(
                gumb_hbm.at[pl.ds(0, _CH)], gb.at[pl.ds(j * _CH, _CH)],
                sg).wait()

    def sweep(lb, gb, jrow, j, tv, m_vec, base, nv, carry, gstride):
        def body(i, carry1):
            mv, sb, si = carry1
            sl_ = pl.ds(i * 16, 16)
            l = lb[jrow, sl_]
            a = l + tv * gb[pl.ds(j * gstride + i * 16, 16)]
            s = jnp.exp(a - m_vec)
            vidx = (base + i * 16) + lane
            upd = s > sb
            return (jnp.maximum(mv, a),
                    jnp.where(upd, s, sb),
                    jnp.where(upd, vidx, si))
        return lax.fori_loop(0, nv, body, carry, unroll=5)

    def proc(c, par, st, tail=False):
        if tail:
            lb, gb = ltail, gtail
            base = _NFULL * _CH
            nv = _NVT
        else:
            lb, _ = lbufs[par]
            gb, _ = gbufs[par]
            base = c * _CH
            nv = _NV
        new_st = []
        for j in range(_RPW):
            m_norm, sbest, sidx = st[j]
            row = wid * _RPW + j
            tv = plsc.load_gather(tbuf, [jnp.full((16,), row, jnp.int32)])
            jrow = 4 * half + j
            mv0 = jnp.full((16,), neg_inf, jnp.float32)
            mv, sb1, si1 = sweep(
                lb, gb, jrow, j, tv, jnp.full((16,), m_norm, jnp.float32),
                base, nv, (mv0, sbest, sidx), nv * 16)
            m_new = jnp.max(mv)

            def no_redo(_, sb1=sb1, si1=si1, m_norm=m_norm):
                return m_norm, sb1, si1

            def redo(_, lb=lb, gb=gb, jrow=jrow, j=j, tv=tv, base=base,
                     nv=nv, mv=mv, m_new=m_new, m_norm=m_norm, sbest=sbest,
                     sidx=sidx):
                sb0 = sbest * jnp.exp(
                    jnp.full((16,), m_norm - m_new, jnp.float32))
                _, sb2, si2 = sweep(
                    lb, gb, jrow, j, tv,
                    jnp.full((16,), m_new, jnp.float32),
                    base, nv, (mv, sb0, sidx), nv * 16)
                return m_new, sb2, si2

            m2, sb2, si2 = lax.cond(m_new > m_norm + jnp.float32(80.0),
                                    redo, no_redo, 0)
            new_st.append((m2, sb2, si2))
        return tuple(new_st)

    st = tuple((neg_inf,
                jnp.zeros((16,), jnp.float32), jnp.zeros((16,), jnp.int32))
               for _ in range(_RPW))

    issue(0, 0)
    issue(1, 1)

    def loop_body(k, st):
        c0 = 2 * k
        wait_chunk(c0, 0)
        st = proc(c0, 0, st)
        issue(c0 + 2, 0)
        wait_chunk(c0 + 1, 1)
        st = proc(c0 + 1, 1, st)
        issue(c0 + 3, 1)
        return st

    st = lax.fori_loop(0, 15, loop_body, st)

    wait_chunk(30, 0)
    st = proc(30, 0, st)
    issue(32, 0)
    wait_chunk(31, 1)
    st = proc(31, 1, st)
    tcol = _NFULL * _CH
    pltpu.async_copy(logits_hbm.at[pl.ds(grp, 8), pl.ds(tcol, _TAIL)],
                     ltail, slt)
    for j in range(_RPW):
        toff = pl.multiple_of((wid * _RPW + j) * _V + tcol, 8)
        pltpu.async_copy(gumb_hbm.at[pl.ds(toff, _TAIL)],
                         gtail.at[pl.ds(j * _TAIL, _TAIL)], sgt)
    wait_chunk(32, 0)
    st = proc(32, 0, st)
    pltpu.make_async_copy(logits_hbm.at[pl.ds(grp, 8), pl.ds(tcol, _TAIL)],
                          ltail, slt).wait()
    for j in range(_RPW):
        pltpu.make_async_copy(gumb_hbm.at[pl.ds(0, _TAIL)],
                              gtail.at[pl.ds(j * _TAIL, _TAIL)], sgt).wait()
    st = proc(0, 0, st, tail=True)

    tokens = jnp.zeros((16,), jnp.int32)
    ibig = jnp.full((16,), _IBIG, jnp.int32)
    for j in range(_RPW):
        _, sbest, sidx = st[j]
        smax = jnp.full((16,), jnp.max(sbest), jnp.float32)
        stok = jnp.min(jnp.where(sbest == smax, sidx, ibig))
        tokens = jnp.where(lane == j, jnp.full((16,), stok, jnp.int32),
                           tokens)

    obuf[...] = tokens
    pltpu.sync_copy(obuf, out_hbm.at[pl.ds(wid * _OUTP, _OUTP)])


_sampler_cache = []


def _sampler_sc():
    if not _sampler_cache:
        _sampler_cache.append(pl.kernel(
            _sampler_sc_body,
            out_type=jax.ShapeDtypeStruct((_NW * _OUTP,), jnp.int32),
            mesh=plsc.VectorSubcoreMesh(core_axis_name="c",
                                        subcore_axis_name="s",
                                        num_cores=_NC, num_subcores=_NS),
            scratch_types=[
                pltpu.VMEM((8, _CH), jnp.float32),
                pltpu.VMEM((_RPW * _CH,), jnp.float32),
                pltpu.VMEM((8, _CH), jnp.float32),
                pltpu.VMEM((_RPW * _CH,), jnp.float32),
                pltpu.VMEM((8, _TAIL), jnp.float32),
                pltpu.VMEM((_RPW * _TAIL,), jnp.float32),
                pltpu.VMEM((_B,), jnp.float32),
                pltpu.VMEM((_OUTP,), jnp.int32),
                pltpu.SemaphoreType.DMA,
                pltpu.SemaphoreType.DMA,
                pltpu.SemaphoreType.DMA,
                pltpu.SemaphoreType.DMA,
                pltpu.SemaphoreType.DMA,
                pltpu.SemaphoreType.DMA,
            ],
            compiler_params=pltpu.CompilerParams(needs_layout_passes=False),
        ))
    return _sampler_cache[0]


def kernel(logits, temperatures):
    gumb = jnp.asarray(_noise_recip())
    flat = _sampler_sc()(logits, gumb, temperatures)
    return flat.reshape(_NW, _OUTP)[:, :_RPW].reshape(_B)
